# baseline (speedup 1.0000x reference)
import jax
import jax.numpy as jnp
import numpy as np
from jax import lax
from jax.experimental import pallas as pl
from jax.experimental.pallas import tpu as pltpu
from jax.experimental.pallas import tpu_sc as plsc

D = 64
SEQ = 200
BATCH = 1024
SCALE = float(np.sqrt(D))

NC = 1
NS = 16
NW = NC * NS
ROWS = BATCH * SEQ
ROWS_W = ROWS // NW
CH = 2 * SEQ
NCH = ROWS_W // CH
NBUF = 2


def _make_pe():
    pos = np.arange(SEQ, dtype=np.float64)[:, None]
    i_even = np.arange(0, D, 2, dtype=np.float64)
    pe = np.zeros((SEQ, D), dtype=np.float64)
    pe[:, 0::2] = np.sin(pos / (10000.0 ** (2.0 * i_even / D)))
    pe[:, 1::2] = np.cos(pos / (10000.0 ** (2.0 * (i_even + 1.0) / D)))
    return pe.astype(np.float32)


_PE = _make_pe()


def _sc_body(table, idxf, pe, out, idx_v, pe_v, in_bufs, out_bufs, gsems, ssems):
    c = lax.axis_index("c")
    s = lax.axis_index("s")
    wid = s * NC + c
    rbase = wid * ROWS_W

    pltpu.sync_copy(idxf.at[pl.ds(rbase, ROWS_W)], idx_v)
    pltpu.sync_copy(pe, pe_v)

    for b in range(NBUF):
        pltpu.async_copy(table.at[idx_v.at[pl.ds(b * CH, CH)]],
                         in_bufs[b], gsems[b])

    @pl.loop(0, NCH, step=NBUF)
    def _chunks(g):
        for b in range(NBUF):
            cidx = g + b

            pltpu.make_async_copy(table.at[pl.ds(0, CH)],
                                  in_bufs[b], gsems[b]).wait()

            @pl.when(cidx >= NBUF)
            def _():
                pltpu.make_async_copy(out_bufs[b],
                                      out.at[pl.ds(0, CH)], ssems[b]).wait()

            for h in range(CH // SEQ):
                @pl.loop(0, SEQ)
                def _rows(r):
                    for q in range(D // 16):
                        sl = pl.ds(q * 16, 16)
                        out_bufs[b][h * SEQ + r, sl] = (
                            in_bufs[b][h * SEQ + r, sl] * SCALE + pe_v[r, sl])

            @pl.when(cidx + NBUF < NCH)
            def _():
                pltpu.async_copy(
                    table.at[idx_v.at[pl.ds((cidx + NBUF) * CH, CH)]],
                    in_bufs[b], gsems[b])

            pltpu.async_copy(out_bufs[b],
                             out.at[pl.ds(rbase + cidx * CH, CH)], ssems[b])

    for b in range(NBUF):
        pltpu.make_async_copy(out_bufs[b], out.at[pl.ds(0, CH)],
                              ssems[b]).wait()


def kernel(x, embed_table):
    pe = jnp.asarray(_PE)
    idx_flat = x.reshape(ROWS)

    mesh = plsc.VectorSubcoreMesh(core_axis_name="c", subcore_axis_name="s",
                                  num_cores=NC, num_subcores=NS)

    def body(table, idxf, pe_in, out, idx_v, pe_v,
             in0, in1, out0, out1, gs0, gs1, ss0, ss1):
        _sc_body(table, idxf, pe_in, out, idx_v, pe_v,
                 (in0, in1), (out0, out1), (gs0, gs1), (ss0, ss1))

    out_flat = pl.kernel(
        body,
        out_type=jax.ShapeDtypeStruct((ROWS, D), jnp.float32),
        mesh=mesh,
        compiler_params=pltpu.CompilerParams(use_tc_tiling_on_sc=False),
        scratch_types=[
            pltpu.VMEM((ROWS_W,), jnp.int32),
            pltpu.VMEM((SEQ, D), jnp.float32),
            pltpu.VMEM((CH, D), jnp.float32),
            pltpu.VMEM((CH, D), jnp.float32),
            pltpu.VMEM((CH, D), jnp.float32),
            pltpu.VMEM((CH, D), jnp.float32),
            pltpu.SemaphoreType.DMA,
            pltpu.SemaphoreType.DMA,
            pltpu.SemaphoreType.DMA,
            pltpu.SemaphoreType.DMA,
        ],
    )(embed_table, idx_flat, pe)

    return out_flat.reshape(BATCH, SEQ, D)


# speedup vs baseline: 1.0327x; 1.0327x over previous
import jax
import jax.numpy as jnp
import numpy as np
from jax import lax
from jax.experimental import pallas as pl
from jax.experimental.pallas import tpu as pltpu
from jax.experimental.pallas import tpu_sc as plsc

D = 64
SEQ = 200
BATCH = 1024
SCALE = float(np.sqrt(D))

NC = 2
NS = 16
NW = NC * NS
ROWS = BATCH * SEQ
ROWS_W = ROWS // NW
CH = 2 * SEQ
NCH = ROWS_W // CH
NBUF = 2


def _make_pe():
    pos = np.arange(SEQ, dtype=np.float64)[:, None]
    i_even = np.arange(0, D, 2, dtype=np.float64)
    pe = np.zeros((SEQ, D), dtype=np.float64)
    pe[:, 0::2] = np.sin(pos / (10000.0 ** (2.0 * i_even / D)))
    pe[:, 1::2] = np.cos(pos / (10000.0 ** (2.0 * (i_even + 1.0) / D)))
    return pe.astype(np.float32)


_PE = _make_pe()


def _sc_body(table, idxf, pe, out, idx_v, pe_v, in_bufs, out_bufs, gsems, ssems):
    c = lax.axis_index("c")
    s = lax.axis_index("s")
    wid = s * NC + c
    rbase = wid * ROWS_W

    pltpu.sync_copy(idxf.at[pl.ds(rbase, ROWS_W)], idx_v)
    pltpu.sync_copy(pe, pe_v)

    for b in range(NBUF):
        pltpu.async_copy(table.at[idx_v.at[pl.ds(b * CH, CH)]],
                         in_bufs[b], gsems[b])

    @pl.loop(0, NCH, step=NBUF)
    def _chunks(g):
        for b in range(NBUF):
            cidx = g + b

            pltpu.make_async_copy(table.at[pl.ds(0, CH)],
                                  in_bufs[b], gsems[b]).wait()

            @pl.when(cidx >= NBUF)
            def _():
                pltpu.make_async_copy(out_bufs[b],
                                      out.at[pl.ds(0, CH)], ssems[b]).wait()

            for h in range(CH // SEQ):
                @pl.loop(0, SEQ)
                def _rows(r):
                    for q in range(D // 16):
                        sl = pl.ds(q * 16, 16)
                        out_bufs[b][h * SEQ + r, sl] = (
                            in_bufs[b][h * SEQ + r, sl] * SCALE + pe_v[r, sl])

            @pl.when(cidx + NBUF < NCH)
            def _():
                pltpu.async_copy(
                    table.at[idx_v.at[pl.ds((cidx + NBUF) * CH, CH)]],
                    in_bufs[b], gsems[b])

            pltpu.async_copy(out_bufs[b],
                             out.at[pl.ds(rbase + cidx * CH, CH)], ssems[b])

    for b in range(NBUF):
        pltpu.make_async_copy(out_bufs[b], out.at[pl.ds(0, CH)],
                              ssems[b]).wait()


def kernel(x, embed_table):
    pe = jnp.asarray(_PE)
    idx_flat = x.reshape(ROWS)

    mesh = plsc.VectorSubcoreMesh(core_axis_name="c", subcore_axis_name="s",
                                  num_cores=NC, num_subcores=NS)

    def body(table, idxf, pe_in, out, idx_v, pe_v,
             in0, in1, out0, out1, gs0, gs1, ss0, ss1):
        _sc_body(table, idxf, pe_in, out, idx_v, pe_v,
                 (in0, in1), (out0, out1), (gs0, gs1), (ss0, ss1))

    out_flat = pl.kernel(
        body,
        out_type=jax.ShapeDtypeStruct((ROWS, D), jnp.float32),
        mesh=mesh,
        compiler_params=pltpu.CompilerParams(use_tc_tiling_on_sc=False),
        scratch_types=[
            pltpu.VMEM((ROWS_W,), jnp.int32),
            pltpu.VMEM((SEQ, D), jnp.float32),
            pltpu.VMEM((CH, D), jnp.float32),
            pltpu.VMEM((CH, D), jnp.float32),
            pltpu.VMEM((CH, D), jnp.float32),
            pltpu.VMEM((CH, D), jnp.float32),
            pltpu.SemaphoreType.DMA,
            pltpu.SemaphoreType.DMA,
            pltpu.SemaphoreType.DMA,
            pltpu.SemaphoreType.DMA,
        ],
    )(embed_table, idx_flat, pe)

    return out_flat.reshape(BATCH, SEQ, D)
